# baseline (speedup 1.0000x reference)
import jax
import jax.numpy as jnp
from jax.experimental import pallas as pl

N_REL = 1000
BB = 256


def _scale_kernel(rel_ref, table_ref, node_ref, out_ref):
    idx = rel_ref[0, 0, :]
    onehot = (idx[:, None] == jax.lax.iota(jnp.int32, N_REL)[None, :]).astype(
        jnp.float32
    )
    r = jax.lax.dot(
        onehot, table_ref[...], precision=jax.lax.Precision.HIGHEST
    )
    r = jax.nn.sigmoid(r)
    out_ref[...] = node_ref[...] * r[:, None, :]


def kernel(node_emb, relation, rela_emb_):
    batch, hist, emb = node_emb.shape
    num_blocks = batch // BB
    rel3 = relation.astype(jnp.int32).reshape(num_blocks, 1, BB)
    return pl.pallas_call(
        _scale_kernel,
        grid=(num_blocks,),
        in_specs=[
            pl.BlockSpec((1, 1, BB), lambda i: (i, 0, 0)),
            pl.BlockSpec((N_REL, emb), lambda i: (0, 0)),
            pl.BlockSpec((BB, hist, emb), lambda i: (i, 0, 0)),
        ],
        out_specs=pl.BlockSpec((BB, hist, emb), lambda i: (i, 0, 0)),
        out_shape=jax.ShapeDtypeStruct((batch, hist, emb), node_emb.dtype),
    )(rel3, rela_emb_, node_emb)


# speedup vs baseline: 1.0324x; 1.0324x over previous
import jax
import jax.numpy as jnp
from jax.experimental import pallas as pl
from jax.experimental.pallas import tpu as pltpu

N_REL = 1000
CH = 128
NBUF = 6


def _body(rel_ref, table_ref, node_hbm, out_hbm, node_buf, out_buf, in_sems, out_sems):
    nchunks = rel_ref.shape[0]
    iota = jax.lax.iota(jnp.int32, N_REL)[None, :]

    def in_copy(k):
        s = k % NBUF
        return pltpu.make_async_copy(
            node_hbm.at[pl.ds(k * CH, CH)], node_buf.at[s], in_sems.at[s]
        )

    def out_copy(k):
        s = k % NBUF
        return pltpu.make_async_copy(
            out_buf.at[s], out_hbm.at[pl.ds(k * CH, CH)], out_sems.at[s]
        )

    for k in range(min(NBUF, nchunks)):
        in_copy(k).start()

    for k in range(nchunks):
        s = k % NBUF
        in_copy(k).wait()
        if k >= NBUF:
            out_copy(k - NBUF).wait()
        idx = rel_ref[k, :]
        onehot = (idx[:, None] == iota).astype(jnp.float32)
        r = jax.lax.dot(onehot, table_ref[...], precision=jax.lax.Precision.HIGHEST)
        r = jax.nn.sigmoid(r)
        out_buf[s] = node_buf[s] * r[:, None, :]
        out_copy(k).start()
        if k + NBUF < nchunks:
            in_copy(k + NBUF).start()

    for k in range(max(0, nchunks - NBUF), nchunks):
        out_copy(k).wait()


def kernel(node_emb, relation, rela_emb_):
    batch, hist, emb = node_emb.shape
    nchunks = batch // CH
    rel2 = relation.astype(jnp.int32).reshape(nchunks, CH)
    return pl.pallas_call(
        _body,
        in_specs=[
            pl.BlockSpec(memory_space=pltpu.MemorySpace.VMEM),
            pl.BlockSpec(memory_space=pltpu.MemorySpace.VMEM),
            pl.BlockSpec(memory_space=pltpu.MemorySpace.HBM),
        ],
        out_specs=pl.BlockSpec(memory_space=pltpu.MemorySpace.HBM),
        out_shape=jax.ShapeDtypeStruct((batch, hist, emb), node_emb.dtype),
        scratch_shapes=[
            pltpu.VMEM((NBUF, CH, hist, emb), jnp.float32),
            pltpu.VMEM((NBUF, CH, hist, emb), jnp.float32),
            pltpu.SemaphoreType.DMA((NBUF,)),
            pltpu.SemaphoreType.DMA((NBUF,)),
        ],
    )(rel2, rela_emb_, node_emb)


# speedup vs baseline: 1.7852x; 1.7293x over previous
import jax
import jax.numpy as jnp
from jax.experimental import pallas as pl
from jax.experimental.pallas import tpu as pltpu

N_REL = 1000
CH = 128
NBUF = 6


def _body(rel_ref, table_ref, node_hbm, out_hbm, node_buf, out_buf, in_sems, out_sems):
    nchunks = rel_ref.shape[0]
    width = node_buf.shape[-1]
    iota = jax.lax.iota(jnp.int32, N_REL)[None, :]

    def in_copy(k):
        s = k % NBUF
        return pltpu.make_async_copy(
            node_hbm.at[pl.ds(k * CH, CH)], node_buf.at[s], in_sems.at[s]
        )

    def out_copy(k):
        s = k % NBUF
        return pltpu.make_async_copy(
            out_buf.at[s], out_hbm.at[pl.ds(k * CH, CH)], out_sems.at[s]
        )

    for k in range(min(NBUF, nchunks)):
        in_copy(k).start()

    for k in range(nchunks):
        s = k % NBUF
        in_copy(k).wait()
        if k >= NBUF:
            out_copy(k - NBUF).wait()
        idx = rel_ref[k, :]
        onehot = (idx[:, None] == iota).astype(jnp.float32)
        r = jax.lax.dot(onehot, table_ref[...], precision=jax.lax.Precision.HIGHEST)
        r = jax.nn.sigmoid(r)
        r2 = jnp.concatenate([r, r], axis=1)
        for j in range(width // 128):
            sl = slice(128 * j, 128 * (j + 1))
            out_buf[s, :, sl] = node_buf[s, :, sl] * r2
        out_copy(k).start()
        if k + NBUF < nchunks:
            in_copy(k + NBUF).start()

    for k in range(max(0, nchunks - NBUF), nchunks):
        out_copy(k).wait()


def kernel(node_emb, relation, rela_emb_):
    batch, hist, emb = node_emb.shape
    nchunks = batch // CH
    rel2 = relation.astype(jnp.int32).reshape(nchunks, CH)
    node2d = node_emb.reshape(batch, hist * emb)
    out2d = pl.pallas_call(
        _body,
        in_specs=[
            pl.BlockSpec(memory_space=pltpu.MemorySpace.VMEM),
            pl.BlockSpec(memory_space=pltpu.MemorySpace.VMEM),
            pl.BlockSpec(memory_space=pltpu.MemorySpace.HBM),
        ],
        out_specs=pl.BlockSpec(memory_space=pltpu.MemorySpace.HBM),
        out_shape=jax.ShapeDtypeStruct((batch, hist * emb), node_emb.dtype),
        scratch_shapes=[
            pltpu.VMEM((NBUF, CH, hist * emb), jnp.float32),
            pltpu.VMEM((NBUF, CH, hist * emb), jnp.float32),
            pltpu.SemaphoreType.DMA((NBUF,)),
            pltpu.SemaphoreType.DMA((NBUF,)),
        ],
    )(rel2, rela_emb_, node2d)
    return out2d.reshape(batch, hist, emb)
